# baseline (speedup 1.0000x reference)
import functools

import jax
import jax.numpy as jnp
from jax import lax
from jax.experimental import pallas as pl
from jax.experimental.pallas import tpu as pltpu
from jax.experimental.pallas import tpu_sc as plsc

N_CAT = 1000
L = 16
NC = 2
NS = 16
NW = NC * NS
CI = 1
NBUF = 2


def _one_hot_sc(x_flat, n0, n1):
    rows_w = n0 // NW
    n_chunks = rows_w // CI
    rpc = CI * n1
    mesh = plsc.VectorSubcoreMesh(core_axis_name="c", subcore_axis_name="s")

    @functools.partial(
        pl.kernel,
        out_type=jax.ShapeDtypeStruct((n0, n1, N_CAT), jnp.float32),
        mesh=mesh,
        scratch_types=[
            pltpu.VMEM((rows_w * n1,), jnp.int32),
            [pltpu.VMEM((CI, n1, N_CAT), jnp.float32)] * NBUF,
            [pltpu.SemaphoreType.DMA] * NBUF,
        ],
        compiler_params=pltpu.CompilerParams(
            needs_layout_passes=False, use_tc_tiling_on_sc=True
        ),
    )
    def body(x_hbm, out_hbm, idx_v, bufs, sems):
        wid = lax.axis_index("s") * NC + lax.axis_index("c")
        i_base = wid * rows_w

        pltpu.sync_copy(x_hbm.at[pl.ds(wid * rows_w * n1, rows_w * n1)], idx_v)

        zeros = jnp.zeros((L,), jnp.float32)
        ones = jnp.ones((L,), jnp.float32)
        lane = lax.iota(jnp.int32, L)

        def zero_body(w, _):
            p = w * L + lane
            idxs = [p // (n1 * N_CAT), (p // N_CAT) % n1, p % N_CAT]
            for b in range(NBUF):
                plsc.store_scatter(bufs[b], idxs, zeros)
            return 0

        lax.fori_loop(0, (CI * n1 * N_CAT) // L, zero_body, 0)

        n_full, tail = divmod(rpc, L)

        def scatter_vals(b, k, vals):
            for g in range(n_full + (1 if tail else 0)):
                cols = idx_v[pl.ds(k * rpc + g * L, L)]
                r = g * L + lane
                idxs = [r // n1, r % n1, cols]
                if g < n_full:
                    plsc.store_scatter(bufs[b], idxs, vals)
                else:
                    plsc.store_scatter(bufs[b], idxs, vals, mask=lane < tail)

        def dma(b, k):
            return pltpu.make_async_copy(
                bufs[b], out_hbm.at[pl.ds(i_base + k * CI, CI)], sems[b]
            )

        for b in range(NBUF):
            scatter_vals(b, b, ones)
            dma(b, b).start()

        def group_body(g, _):
            for b in range(NBUF):
                k = g * NBUF + b
                dma(b, k - NBUF).wait()
                scatter_vals(b, k - NBUF, zeros)
                scatter_vals(b, k, ones)
                dma(b, k).start()
            return 0

        lax.fori_loop(1, n_chunks // NBUF, group_body, 0)

        for b in range(NBUF):
            dma(b, n_chunks - NBUF + b).wait()

    return body(x_flat)


def kernel(x, eye):
    n0, n1 = x.shape
    x_flat = x.astype(jnp.int32).reshape(n0 * n1)
    return _one_hot_sc(x_flat, n0, n1)
